# Initial kernel scaffold; baseline (speedup 1.0000x reference)
import jax
import jax.numpy as jnp
from jax.experimental import pallas as pl


def kernel(x, router_w):
    raise NotImplementedError("write your pallas kernel here")


# speedup vs baseline: 4.1591x; 4.1591x over previous
import functools

import jax
import jax.numpy as jnp
from jax.experimental import pallas as pl
from jax.experimental.pallas import tpu as pltpu

_N_TOKENS = 8192
_DIM = 2048
_N_EXPERTS = 64
_N_GROUPS = 2
_GROUP_SIZE = _N_EXPERTS // _N_GROUPS
_BLOCK = 512

_NEG_INF = float("-inf")


def _gate_block(x_ref, wt_ref, w_out_ref, i_out_ref):
    s = jnp.dot(x_ref[...], wt_ref[...], preferred_element_type=jnp.float32)
    m = jnp.max(s, axis=-1, keepdims=True)
    e = jnp.exp(s - m)
    probs = e / jnp.sum(e, axis=-1, keepdims=True)

    lane = jax.lax.broadcasted_iota(jnp.int32, probs.shape, 1)
    g0 = jnp.max(jnp.where(lane < _GROUP_SIZE, probs, _NEG_INF), axis=-1, keepdims=True)
    g1 = jnp.max(jnp.where(lane >= _GROUP_SIZE, probs, _NEG_INF), axis=-1, keepdims=True)
    gsel = jnp.where(g1 > g0, 1, 0)
    lane_group = lane // _GROUP_SIZE
    sm = jnp.where(lane_group == gsel, probs, _NEG_INF)

    m1 = jnp.max(sm, axis=-1, keepdims=True)
    i1 = jnp.min(jnp.where(sm == m1, lane, _N_EXPERTS), axis=-1, keepdims=True)
    sm2 = jnp.where(lane == i1, _NEG_INF, sm)
    m2 = jnp.max(sm2, axis=-1, keepdims=True)
    i2 = jnp.min(jnp.where(sm2 == m2, lane, _N_EXPERTS), axis=-1, keepdims=True)

    w_out_ref[...] = jnp.concatenate([m1, m2], axis=-1)
    i_out_ref[...] = jnp.concatenate([i1, i2], axis=-1)


@jax.jit
def kernel(x, router_w):
    n = x.shape[0]
    grid = (n // _BLOCK,)
    wt = router_w.T
    weights, indices = pl.pallas_call(
        _gate_block,
        grid=grid,
        in_specs=[
            pl.BlockSpec((_BLOCK, _DIM), lambda i: (i, 0)),
            pl.BlockSpec((_DIM, _N_EXPERTS), lambda i: (0, 0)),
        ],
        out_specs=[
            pl.BlockSpec((_BLOCK, 2), lambda i: (i, 0)),
            pl.BlockSpec((_BLOCK, 2), lambda i: (i, 0)),
        ],
        out_shape=[
            jax.ShapeDtypeStruct((n, 2), jnp.float32),
            jax.ShapeDtypeStruct((n, 2), jnp.int32),
        ],
        compiler_params=pltpu.CompilerParams(
            dimension_semantics=("arbitrary",),
        ),
    )(x, wt)
    return weights, indices


# speedup vs baseline: 4.2284x; 1.0167x over previous
import jax
import jax.numpy as jnp
from jax.experimental import pallas as pl
from jax.experimental.pallas import tpu as pltpu

_DIM = 2048
_N_EXPERTS = 64
_N_GROUPS = 2
_GROUP_SIZE = _N_EXPERTS // _N_GROUPS
_BLOCK = 512

_IDX_MASK = _N_EXPERTS - 1
_KEY_MIN = -2147483647 - 1


def _to_key(s):
    u = jax.lax.bitcast_convert_type(s, jnp.int32)
    return jnp.where(u < 0, u ^ jnp.int32(0x7FFFFFFF), u)


def _from_key(k):
    u = jnp.where(k < 0, k ^ jnp.int32(0x7FFFFFFF), k)
    return jax.lax.bitcast_convert_type(u, jnp.float32)


def _gate_block(x_ref, wt_ref, w_out_ref, i_out_ref):
    s = jnp.dot(x_ref[...], wt_ref[...], preferred_element_type=jnp.float32)

    lane = jax.lax.broadcasted_iota(jnp.int32, s.shape, 1)
    key = (_to_key(s) & ~jnp.int32(_IDX_MASK)) | (_IDX_MASK - lane)

    kg0 = jnp.max(jnp.where(lane < _GROUP_SIZE, key, _KEY_MIN), axis=-1, keepdims=True)
    kg1 = jnp.max(jnp.where(lane >= _GROUP_SIZE, key, _KEY_MIN), axis=-1, keepdims=True)
    gsel = jnp.where(kg1 > kg0, 1, 0)
    km = jnp.where(lane // _GROUP_SIZE == gsel, key, _KEY_MIN)

    k1 = jnp.max(km, axis=-1, keepdims=True)
    k2 = jnp.max(jnp.where(key == k1, _KEY_MIN, km), axis=-1, keepdims=True)

    m = _from_key(jnp.maximum(kg0, kg1) & ~jnp.int32(_IDX_MASK))
    z = jnp.sum(jnp.exp(s - m), axis=-1, keepdims=True)

    v1 = _from_key(k1 & ~jnp.int32(_IDX_MASK))
    v2 = _from_key(k2 & ~jnp.int32(_IDX_MASK))
    w1 = jnp.exp(v1 - m) / z
    w2 = jnp.exp(v2 - m) / z
    i1 = _IDX_MASK - (k1 & _IDX_MASK)
    i2 = _IDX_MASK - (k2 & _IDX_MASK)

    w_out_ref[...] = jnp.concatenate([w1, w2], axis=-1)
    i_out_ref[...] = jnp.concatenate([i1, i2], axis=-1)


@jax.jit
def kernel(x, router_w):
    n = x.shape[0]
    grid = (n // _BLOCK,)
    wt = router_w.T
    weights, indices = pl.pallas_call(
        _gate_block,
        grid=grid,
        in_specs=[
            pl.BlockSpec((_BLOCK, _DIM), lambda i: (i, 0)),
            pl.BlockSpec((_DIM, _N_EXPERTS), lambda i: (0, 0)),
        ],
        out_specs=[
            pl.BlockSpec((_BLOCK, 2), lambda i: (i, 0)),
            pl.BlockSpec((_BLOCK, 2), lambda i: (i, 0)),
        ],
        out_shape=[
            jax.ShapeDtypeStruct((n, 2), jnp.float32),
            jax.ShapeDtypeStruct((n, 2), jnp.int32),
        ],
        compiler_params=pltpu.CompilerParams(
            dimension_semantics=("arbitrary",),
        ),
    )(x, wt)
    return weights, indices


# speedup vs baseline: 4.6689x; 1.1042x over previous
import jax
import jax.numpy as jnp
from jax.experimental import pallas as pl
from jax.experimental.pallas import tpu as pltpu

_DIM = 2048
_N_EXPERTS = 64
_N_GROUPS = 2
_GROUP_SIZE = _N_EXPERTS // _N_GROUPS
_BLOCK = 1024

_IDX_MASK = _N_EXPERTS - 1
_KEY_MIN = -2147483647 - 1


def _to_key(s):
    u = jax.lax.bitcast_convert_type(s, jnp.int32)
    return jnp.where(u < 0, u ^ jnp.int32(0x7FFFFFFF), u)


def _from_key(k):
    u = jnp.where(k < 0, k ^ jnp.int32(0x7FFFFFFF), k)
    return jax.lax.bitcast_convert_type(u, jnp.float32)


def _gate_block(x_ref, wt_ref, w_out_ref, i_out_ref):
    s = jnp.dot(x_ref[...], wt_ref[...], preferred_element_type=jnp.float32)

    lane = jax.lax.broadcasted_iota(jnp.int32, s.shape, 1)
    key = (_to_key(s) & ~jnp.int32(_IDX_MASK)) | (_IDX_MASK - lane)

    kg0 = jnp.max(jnp.where(lane < _GROUP_SIZE, key, _KEY_MIN), axis=-1, keepdims=True)
    kg1 = jnp.max(jnp.where(lane >= _GROUP_SIZE, key, _KEY_MIN), axis=-1, keepdims=True)
    gsel = jnp.where(kg1 > kg0, 1, 0)
    km = jnp.where(lane // _GROUP_SIZE == gsel, key, _KEY_MIN)

    k1 = jnp.max(km, axis=-1, keepdims=True)
    k2 = jnp.max(jnp.where(key == k1, _KEY_MIN, km), axis=-1, keepdims=True)

    m = _from_key(jnp.maximum(kg0, kg1) & ~jnp.int32(_IDX_MASK))
    z = jnp.sum(jnp.exp(s - m), axis=-1, keepdims=True)

    v1 = _from_key(k1 & ~jnp.int32(_IDX_MASK))
    v2 = _from_key(k2 & ~jnp.int32(_IDX_MASK))
    w1 = jnp.exp(v1 - m) / z
    w2 = jnp.exp(v2 - m) / z
    i1 = _IDX_MASK - (k1 & _IDX_MASK)
    i2 = _IDX_MASK - (k2 & _IDX_MASK)

    w_out_ref[...] = jnp.concatenate([w1, w2], axis=-1)
    i_out_ref[...] = jnp.concatenate([i1, i2], axis=-1)


@jax.jit
def kernel(x, router_w):
    n = x.shape[0]
    grid = (n // _BLOCK,)
    wt = router_w.T
    weights, indices = pl.pallas_call(
        _gate_block,
        grid=grid,
        in_specs=[
            pl.BlockSpec((_BLOCK, _DIM), lambda i: (i, 0)),
            pl.BlockSpec((_DIM, _N_EXPERTS), lambda i: (0, 0)),
        ],
        out_specs=[
            pl.BlockSpec((_BLOCK, 2), lambda i: (i, 0)),
            pl.BlockSpec((_BLOCK, 2), lambda i: (i, 0)),
        ],
        out_shape=[
            jax.ShapeDtypeStruct((n, 2), jnp.float32),
            jax.ShapeDtypeStruct((n, 2), jnp.int32),
        ],
        compiler_params=pltpu.CompilerParams(
            dimension_semantics=("arbitrary",),
        ),
    )(x, wt)
    return weights, indices


# speedup vs baseline: 4.6712x; 1.0005x over previous
import jax
import jax.numpy as jnp
from jax.experimental import pallas as pl
from jax.experimental.pallas import tpu as pltpu

_DIM = 2048
_N_EXPERTS = 64
_N_GROUPS = 2
_GROUP_SIZE = _N_EXPERTS // _N_GROUPS
_BLOCK = 2048

_IDX_MASK = _N_EXPERTS - 1
_KEY_MIN = -2147483647 - 1


def _to_key(s):
    u = jax.lax.bitcast_convert_type(s, jnp.int32)
    return jnp.where(u < 0, u ^ jnp.int32(0x7FFFFFFF), u)


def _from_key(k):
    u = jnp.where(k < 0, k ^ jnp.int32(0x7FFFFFFF), k)
    return jax.lax.bitcast_convert_type(u, jnp.float32)


def _gate_block(x_ref, wt_ref, w_out_ref, i_out_ref):
    s = jnp.dot(x_ref[...], wt_ref[...], preferred_element_type=jnp.float32)

    lane = jax.lax.broadcasted_iota(jnp.int32, s.shape, 1)
    key = (_to_key(s) & ~jnp.int32(_IDX_MASK)) | (_IDX_MASK - lane)

    kg0 = jnp.max(jnp.where(lane < _GROUP_SIZE, key, _KEY_MIN), axis=-1, keepdims=True)
    kg1 = jnp.max(jnp.where(lane >= _GROUP_SIZE, key, _KEY_MIN), axis=-1, keepdims=True)
    gsel = jnp.where(kg1 > kg0, 1, 0)
    km = jnp.where(lane // _GROUP_SIZE == gsel, key, _KEY_MIN)

    k1 = jnp.max(km, axis=-1, keepdims=True)
    k2 = jnp.max(jnp.where(key == k1, _KEY_MIN, km), axis=-1, keepdims=True)

    m = _from_key(jnp.maximum(kg0, kg1) & ~jnp.int32(_IDX_MASK))
    z = jnp.sum(jnp.exp(s - m), axis=-1, keepdims=True)

    v1 = _from_key(k1 & ~jnp.int32(_IDX_MASK))
    v2 = _from_key(k2 & ~jnp.int32(_IDX_MASK))
    w1 = jnp.exp(v1 - m) / z
    w2 = jnp.exp(v2 - m) / z
    i1 = _IDX_MASK - (k1 & _IDX_MASK)
    i2 = _IDX_MASK - (k2 & _IDX_MASK)

    w_out_ref[...] = jnp.concatenate([w1, w2], axis=-1)
    i_out_ref[...] = jnp.concatenate([i1, i2], axis=-1)


@jax.jit
def kernel(x, router_w):
    n = x.shape[0]
    grid = (n // _BLOCK,)
    wt = router_w.T
    weights, indices = pl.pallas_call(
        _gate_block,
        grid=grid,
        in_specs=[
            pl.BlockSpec((_BLOCK, _DIM), lambda i: (i, 0)),
            pl.BlockSpec((_DIM, _N_EXPERTS), lambda i: (0, 0)),
        ],
        out_specs=[
            pl.BlockSpec((_BLOCK, 2), lambda i: (i, 0)),
            pl.BlockSpec((_BLOCK, 2), lambda i: (i, 0)),
        ],
        out_shape=[
            jax.ShapeDtypeStruct((n, 2), jnp.float32),
            jax.ShapeDtypeStruct((n, 2), jnp.int32),
        ],
        compiler_params=pltpu.CompilerParams(
            dimension_semantics=("arbitrary",),
        ),
    )(x, wt)
    return weights, indices
